# baseline (speedup 1.0000x reference)
import functools

import jax
import jax.numpy as jnp
from jax import lax
from jax.experimental import pallas as pl
from jax.experimental.pallas import tpu as pltpu
from jax.experimental.pallas import tpu_sc as plsc

_NUM_CORES = 2
_NUM_SUBCORES = 16
_NUM_WORKERS = _NUM_CORES * _NUM_SUBCORES

_CHUNK = 128


@functools.partial(jax.jit, static_argnums=(2, 3))
def _gather_call(x, index, b, d):
    chunks_per_worker = b // (_NUM_WORKERS * _CHUNK)
    rows_per_worker = b // _NUM_WORKERS
    mesh = plsc.VectorSubcoreMesh(
        core_axis_name="c",
        subcore_axis_name="s",
        num_cores=_NUM_CORES,
        num_subcores=_NUM_SUBCORES,
    )

    @functools.partial(
        pl.kernel,
        out_type=jax.ShapeDtypeStruct((b, d), jnp.float32),
        mesh=mesh,
        compiler_params=pltpu.CompilerParams(use_tc_tiling_on_sc=False),
        scratch_types=[
            pltpu.VMEM((_CHUNK,), jnp.int32),
            pltpu.VMEM((_CHUNK, d), jnp.float32),
            pltpu.SemaphoreType.DMA,
        ],
    )
    def gather_kernel(x_hbm, idx_hbm, out_hbm, idx_v, rows_v, gsem):
        wid = lax.axis_index("s") * _NUM_CORES + lax.axis_index("c")
        base = wid * rows_per_worker

        @pl.loop(0, chunks_per_worker)
        def _chunk_loop(g):
            off = base + g * _CHUNK
            pltpu.sync_copy(idx_hbm.at[pl.ds(off, _CHUNK)], idx_v)
            pltpu.async_copy(x_hbm.at[idx_v], rows_v, gsem).wait()
            pltpu.sync_copy(rows_v, out_hbm.at[pl.ds(off, _CHUNK)])

    return gather_kernel(x, index)


def kernel(x, dim, index):
    del dim
    b = index.shape[0]
    d = x.shape[1]
    return _gather_call(x, index, b, d)


# speedup vs baseline: 1.1256x; 1.1256x over previous
import functools

import jax
import jax.numpy as jnp
from jax import lax
from jax.experimental import pallas as pl
from jax.experimental.pallas import tpu as pltpu
from jax.experimental.pallas import tpu_sc as plsc

_NUM_CORES = 2
_NUM_SUBCORES = 16
_NUM_WORKERS = _NUM_CORES * _NUM_SUBCORES

_CHUNK = 832


@functools.partial(jax.jit, static_argnums=(2, 3))
def _gather_call(x, index, b, d):
    rows_per_worker = b // _NUM_WORKERS
    nch = rows_per_worker // _CHUNK
    mesh = plsc.VectorSubcoreMesh(
        core_axis_name="c",
        subcore_axis_name="s",
        num_cores=_NUM_CORES,
        num_subcores=_NUM_SUBCORES,
    )

    @functools.partial(
        pl.kernel,
        out_type=jax.ShapeDtypeStruct((b, d), jnp.float32),
        mesh=mesh,
        compiler_params=pltpu.CompilerParams(use_tc_tiling_on_sc=False),
        scratch_types=[
            pltpu.VMEM((_CHUNK,), jnp.int32),
            pltpu.VMEM((_CHUNK,), jnp.int32),
            pltpu.VMEM((_CHUNK, d), jnp.float32),
            pltpu.VMEM((_CHUNK, d), jnp.float32),
            pltpu.SemaphoreType.DMA,
            pltpu.SemaphoreType.DMA,
            pltpu.SemaphoreType.DMA,
            pltpu.SemaphoreType.DMA,
        ],
    )
    def gather_kernel(x_hbm, idx_hbm, out_hbm, idx0, idx1, rows0, rows1,
                      gsem0, gsem1, osem0, osem1):
        wid = lax.axis_index("s") * _NUM_CORES + lax.axis_index("c")
        base = wid * rows_per_worker
        idx_b = (idx0, idx1)
        rows_b = (rows0, rows1)
        gsem_b = (gsem0, gsem1)
        osem_b = (osem0, osem1)

        def load_and_gather(g, slot):
            off = base + g * _CHUNK
            pltpu.sync_copy(idx_hbm.at[pl.ds(off, _CHUNK)], idx_b[slot])
            pltpu.async_copy(x_hbm.at[idx_b[slot]], rows_b[slot], gsem_b[slot])

        load_and_gather(0, 0)
        load_and_gather(1, 1)

        @pl.loop(0, nch // 2 - 1)
        def _ring(h):
            g = 2 * h
            for slot in range(2):
                gcur = g + slot
                pltpu.make_async_copy(
                    x_hbm.at[idx_b[slot]], rows_b[slot], gsem_b[slot]).wait()
                pltpu.async_copy(
                    rows_b[slot],
                    out_hbm.at[pl.ds(base + gcur * _CHUNK, _CHUNK)],
                    osem_b[slot])
                pltpu.make_async_copy(
                    rows_b[slot],
                    out_hbm.at[pl.ds(base + gcur * _CHUNK, _CHUNK)],
                    osem_b[slot]).wait()
                load_and_gather(gcur + 2, slot)

        for slot in range(2):
            gcur = nch - 2 + slot
            pltpu.make_async_copy(
                x_hbm.at[idx_b[slot]], rows_b[slot], gsem_b[slot]).wait()
            pltpu.sync_copy(
                rows_b[slot], out_hbm.at[pl.ds(base + gcur * _CHUNK, _CHUNK)])

    return gather_kernel(x, index)


def kernel(x, dim, index):
    del dim
    b = index.shape[0]
    d = x.shape[1]
    return _gather_call(x, index, b, d)


# speedup vs baseline: 1.3412x; 1.1916x over previous
import functools

import jax
import jax.numpy as jnp
from jax import lax
from jax.experimental import pallas as pl
from jax.experimental.pallas import tpu as pltpu
from jax.experimental.pallas import tpu_sc as plsc

_NUM_CORES = 2
_NUM_SUBCORES = 16
_NUM_WORKERS = _NUM_CORES * _NUM_SUBCORES

_CHUNK = 256
_DP = 128


@functools.partial(jax.jit, static_argnums=(2,))
def _gather_call(x, index, b):
    rows_per_worker = b // _NUM_WORKERS
    nch = rows_per_worker // _CHUNK
    xp = jnp.pad(x, ((0, 0), (0, _DP - x.shape[1])))
    mesh = plsc.VectorSubcoreMesh(
        core_axis_name="c",
        subcore_axis_name="s",
        num_cores=_NUM_CORES,
        num_subcores=_NUM_SUBCORES,
    )

    @functools.partial(
        pl.kernel,
        out_type=jax.ShapeDtypeStruct((b, _DP), jnp.float32),
        mesh=mesh,
        compiler_params=pltpu.CompilerParams(use_tc_tiling_on_sc=True),
        scratch_types=[
            pltpu.VMEM((_CHUNK,), jnp.int32),
            pltpu.VMEM((_CHUNK,), jnp.int32),
            pltpu.VMEM((_CHUNK, _DP), jnp.float32),
            pltpu.VMEM((_CHUNK, _DP), jnp.float32),
            pltpu.SemaphoreType.DMA,
            pltpu.SemaphoreType.DMA,
            pltpu.SemaphoreType.DMA,
            pltpu.SemaphoreType.DMA,
        ],
    )
    def gather_kernel(x_hbm, idx_hbm, out_hbm, idx0, idx1, rows0, rows1,
                      gsem0, gsem1, osem0, osem1):
        wid = lax.axis_index("s") * _NUM_CORES + lax.axis_index("c")
        base = wid * rows_per_worker
        idx_b = (idx0, idx1)
        rows_b = (rows0, rows1)
        gsem_b = (gsem0, gsem1)
        osem_b = (osem0, osem1)

        def load_and_gather(g, slot):
            off = base + g * _CHUNK
            pltpu.sync_copy(idx_hbm.at[pl.ds(off, _CHUNK)], idx_b[slot])
            pltpu.async_copy(x_hbm.at[idx_b[slot]], rows_b[slot], gsem_b[slot])

        load_and_gather(0, 0)
        load_and_gather(1, 1)

        @pl.loop(0, nch // 2 - 1)
        def _ring(h):
            g = 2 * h
            for slot in range(2):
                gcur = g + slot
                pltpu.make_async_copy(
                    x_hbm.at[idx_b[slot]], rows_b[slot], gsem_b[slot]).wait()
                pltpu.async_copy(
                    rows_b[slot],
                    out_hbm.at[pl.ds(base + gcur * _CHUNK, _CHUNK)],
                    osem_b[slot])
                pltpu.make_async_copy(
                    rows_b[slot],
                    out_hbm.at[pl.ds(base + gcur * _CHUNK, _CHUNK)],
                    osem_b[slot]).wait()
                load_and_gather(gcur + 2, slot)

        for slot in range(2):
            gcur = nch - 2 + slot
            pltpu.make_async_copy(
                x_hbm.at[idx_b[slot]], rows_b[slot], gsem_b[slot]).wait()
            pltpu.sync_copy(
                rows_b[slot], out_hbm.at[pl.ds(base + gcur * _CHUNK, _CHUNK)])

    return gather_kernel(xp, index)


def kernel(x, dim, index):
    del dim
    b = index.shape[0]
    d = x.shape[1]
    return _gather_call(x, index, b)[:, :d]
